# baseline (speedup 1.0000x reference)
import functools
import jax
import jax.numpy as jnp
from jax import lax
from jax.experimental import pallas as pl
from jax.experimental.pallas import tpu as pltpu
from jax.experimental.pallas import tpu_sc as plsc

B, M, N, F = 4, 16384, 32768, 256
PAD = 128
MP = M + PAD
NSUB = 16
WPB = 8
JW = N // WPB
NCH1 = JW // 16
CH = 32
DEPTH = 3
NCH3 = JW // CH

_mesh = plsc.VectorSubcoreMesh(core_axis_name="c", subcore_axis_name="s")


@functools.partial(
    pl.kernel,
    out_type=jax.ShapeDtypeStruct((B, N, F), jnp.float32),
    mesh=_mesh,
    compiler_params=pltpu.CompilerParams(needs_layout_passes=False),
    scratch_types=[
        pltpu.VMEM((N,), jnp.int32),
        pltpu.VMEM((M,), jnp.int32),
        pltpu.VMEM((JW,), jnp.int32),
        pltpu.VMEM((4, JW), jnp.int32),
        pltpu.VMEM_SHARED((NSUB, N), jnp.int32),
        pltpu.VMEM((64,), jnp.int32),
        pltpu.HBM((B * MP, F), jnp.float32),
        [pltpu.VMEM((CH, F), jnp.float32)] * DEPTH,
        [pltpu.SemaphoreType.DMA] * DEPTH,
    ],
)
def _upsample_sc(feats_hbm, interp_hbm, up_hbm, out_hbm,
                 pos_v, interp_v, acc_v, tmp4, shared,
                 nbr_v, table, bufs, sems):
    c = lax.axis_index("c")
    s = lax.axis_index("s")
    b = 2 * c + s // WPB
    r = s % WPB
    base = r * JW

    up_dma = pltpu.async_copy(up_hbm.at[b, pl.ds(base, JW)], tmp4.at[3],
                              sems[0])
    in_dma = pltpu.async_copy(interp_hbm.at[b], interp_v, sems[1])
    tb_dma = pltpu.async_copy(
        feats_hbm.at[b, pl.ds(r * (M // WPB), M // WPB)],
        table.at[pl.ds(b * MP + r * (M // WPB), M // WPB)], sems[2])
    m1 = jnp.full((16,), -1, jnp.int32)
    z16 = jnp.zeros((16,), jnp.float32)

    def zrow(i, carry):
        for u in range(F // 16):
            bufs[0][i, pl.ds(u * 16, 16)] = z16
        return carry

    lax.fori_loop(0, 16, zrow, 0)
    zr_dma = pltpu.async_copy(
        bufs[0].at[pl.ds(0, 16)],
        table.at[pl.ds(b * MP + M + r * (PAD // WPB), PAD // WPB)], sems[0])

    def ms(i, carry):
        for u in range(8):
            pos_v[pl.ds(i * 128 + u * 16, 16)] = m1
        return carry

    lax.fori_loop(0, N // 128, ms, 0)
    up_dma.wait()
    in_dma.wait()

    ii = lax.iota(jnp.int32, 16)
    nbr_v[pl.ds(16, 16)] = jnp.full((16,), -1, jnp.int32)
    nbr_v[pl.ds(48, 16)] = jnp.full((16,), -1, jnp.int32)

    def ph1(ci, carry):
        res = []
        for u in range(2):
            idx = tmp4[3, pl.ds(ci * 32 + u * 16, 16)]
            nvec = base + ci * 32 + u * 16 + ii
            comb = (idx << 15) | nvec
            scomb, n_s = plsc.sort_key_val(comb, nvec)
            idx_s = lax.shift_right_logical(scomb, 15)
            nbr_v[pl.ds(u * 32, 16)] = idx_s
            nxt = nbr_v[pl.ds(u * 32 + 1, 16)]
            res.append((idx_s, n_s, idx_s != nxt))
        for idx_s, n_s, is_last in res:
            plsc.store_scatter(pos_v, [idx_s], n_s, mask=is_last)
        return carry

    lax.fori_loop(0, NCH1 // 2, ph1, 0)

    pltpu.sync_copy(pos_v, shared.at[s])
    tb_dma.wait()
    zr_dma.wait()
    plsc.subcore_barrier()

    g0 = (s // WPB) * WPB
    mh = [pltpu.async_copy(shared.at[g0 + k, pl.ds(base, JW)],
                           tmp4.at[k], sems[k % DEPTH])
          for k in range(4)]
    for h in mh:
        h.wait()

    def mg1(i, carry):
        sl = pl.ds(i * 16, 16)
        pos = jnp.maximum(jnp.maximum(tmp4[0, sl], tmp4[1, sl]),
                          jnp.maximum(tmp4[2, sl], tmp4[3, sl]))
        acc_v[sl] = pos
        return carry

    lax.fori_loop(0, JW // 16, mg1, 0)

    mh = [pltpu.async_copy(shared.at[g0 + 4 + k, pl.ds(base, JW)],
                           tmp4.at[k], sems[k % DEPTH])
          for k in range(4)]
    for h in mh:
        h.wait()

    boff = b * MP

    def mp(i, carry):
        sl = pl.ds(i * 16, 16)
        pos = jnp.maximum(jnp.maximum(tmp4[0, sl], tmp4[1, sl]),
                          jnp.maximum(tmp4[2, sl], tmp4[3, sl]))
        pos = jnp.maximum(pos, acc_v[sl])
        cidx = jnp.maximum(pos - M, 0)
        ival = plsc.load_gather(interp_v, [cidx])
        row = jnp.where(pos >= M, ival, pos)
        zrow = M + ((i * 16 + ii) & (PAD - 1))
        row = jnp.where(pos >= 0, row, zrow)
        acc_v[sl] = row + boff
        return carry

    lax.fori_loop(0, JW // 16, mp, 0)

    handles = [None] * DEPTH

    def start(k):
        idx_ref = acc_v.at[pl.ds(k * CH, CH)]
        return pltpu.async_copy(table.at[idx_ref],
                                bufs[k % DEPTH], sems[k % DEPTH])

    for k in range(DEPTH - 1):
        handles[k] = start(k)
    for k in range(NCH3):
        if k + DEPTH - 1 < NCH3:
            handles[(k + DEPTH - 1) % DEPTH] = start(k + DEPTH - 1)
        handles[k % DEPTH].wait()
        pltpu.sync_copy(bufs[k % DEPTH],
                        out_hbm.at[b, pl.ds(base + k * CH, CH)])


def kernel(feats, interpolate_idx, upsample_idx):
    assert feats.shape == (B, M, F) and upsample_idx.shape == (B, N)
    return _upsample_sc(feats, interpolate_idx.astype(jnp.int32),
                        upsample_idx.astype(jnp.int32))


# speedup vs baseline: 5.0904x; 5.0904x over previous
import functools
import jax
import jax.numpy as jnp
from jax import lax
from jax.experimental import pallas as pl
from jax.experimental.pallas import tpu as pltpu
from jax.experimental.pallas import tpu_sc as plsc

B, M, N, F = 4, 16384, 32768, 256
PAD = 128
MP = M + PAD
NSUB = 16
WPB = 8
JW = N // WPB
NCH1 = JW // 16
CH = 32
DEPTH = 3
NCH3 = JW // CH

_mesh = plsc.VectorSubcoreMesh(core_axis_name="c", subcore_axis_name="s")


@functools.partial(
    pl.kernel,
    out_type=jax.ShapeDtypeStruct((B, N, F), jnp.float32),
    mesh=_mesh,
    compiler_params=pltpu.CompilerParams(needs_layout_passes=False),
    scratch_types=[
        pltpu.VMEM((N,), jnp.int32),
        pltpu.VMEM((M,), jnp.int32),
        pltpu.VMEM((JW,), jnp.int32),
        pltpu.VMEM((4, JW), jnp.int32),
        pltpu.VMEM_SHARED((NSUB, N), jnp.int32),
        pltpu.VMEM((64,), jnp.int32),
        [pltpu.VMEM((CH, F), jnp.float32)] * DEPTH,
        [pltpu.SemaphoreType.DMA] * DEPTH,
    ],
)
def _upsample_sc(feats_hbm, interp_hbm, up_hbm, out_hbm,
                 pos_v, interp_v, acc_v, tmp4, shared,
                 nbr_v, bufs, sems):
    c = lax.axis_index("c")
    s = lax.axis_index("s")
    b = 2 * c + s // WPB
    r = s % WPB
    base = r * JW

    up_dma = pltpu.async_copy(up_hbm.at[b, pl.ds(base, JW)], tmp4.at[3],
                              sems[0])
    in_dma = pltpu.async_copy(interp_hbm.at[b], interp_v, sems[1])
    m1 = jnp.full((16,), -1, jnp.int32)

    def ms(i, carry):
        for u in range(8):
            pos_v[pl.ds(i * 128 + u * 16, 16)] = m1
        return carry

    lax.fori_loop(0, N // 128, ms, 0)
    up_dma.wait()
    in_dma.wait()

    ii = lax.iota(jnp.int32, 16)
    nbr_v[pl.ds(16, 16)] = jnp.full((16,), -1, jnp.int32)
    nbr_v[pl.ds(48, 16)] = jnp.full((16,), -1, jnp.int32)

    def ph1(ci, carry):
        res = []
        for u in range(2):
            idx = tmp4[3, pl.ds(ci * 32 + u * 16, 16)]
            nvec = base + ci * 32 + u * 16 + ii
            comb = (idx << 15) | nvec
            scomb, n_s = plsc.sort_key_val(comb, nvec)
            idx_s = lax.shift_right_logical(scomb, 15)
            nbr_v[pl.ds(u * 32, 16)] = idx_s
            nxt = nbr_v[pl.ds(u * 32 + 1, 16)]
            res.append((idx_s, n_s, idx_s != nxt))
        for idx_s, n_s, is_last in res:
            plsc.store_scatter(pos_v, [idx_s], n_s, mask=is_last)
        return carry

    lax.fori_loop(0, NCH1 // 2, ph1, 0)

    pltpu.sync_copy(pos_v, shared.at[s])
    plsc.subcore_barrier()

    g0 = (s // WPB) * WPB
    mh = [pltpu.async_copy(shared.at[g0 + k, pl.ds(base, JW)],
                           tmp4.at[k], sems[k % DEPTH])
          for k in range(4)]
    for h in mh:
        h.wait()

    def mg1(i, carry):
        sl = pl.ds(i * 16, 16)
        pos = jnp.maximum(jnp.maximum(tmp4[0, sl], tmp4[1, sl]),
                          jnp.maximum(tmp4[2, sl], tmp4[3, sl]))
        acc_v[sl] = pos
        return carry

    lax.fori_loop(0, JW // 16, mg1, 0)

    mh = [pltpu.async_copy(shared.at[g0 + 4 + k, pl.ds(base, JW)],
                           tmp4.at[k], sems[k % DEPTH])
          for k in range(4)]
    for h in mh:
        h.wait()

    boff = b * MP

    def mp(i, carry):
        sl = pl.ds(i * 16, 16)
        pos = jnp.maximum(jnp.maximum(tmp4[0, sl], tmp4[1, sl]),
                          jnp.maximum(tmp4[2, sl], tmp4[3, sl]))
        pos = jnp.maximum(pos, acc_v[sl])
        cidx = jnp.maximum(pos - M, 0)
        ival = plsc.load_gather(interp_v, [cidx])
        row = jnp.where(pos >= M, ival, pos)
        zrow = M + ((i * 16 + ii) & (PAD - 1))
        row = jnp.where(pos >= 0, row, zrow)
        acc_v[sl] = row + boff
        return carry

    lax.fori_loop(0, JW // 16, mp, 0)

    handles = [None] * DEPTH

    def start(k):
        idx_ref = acc_v.at[pl.ds(k * CH, CH)]
        return pltpu.async_copy(feats_hbm.at[idx_ref],
                                bufs[k % DEPTH], sems[k % DEPTH])

    for k in range(DEPTH - 1):
        handles[k] = start(k)
    for k in range(NCH3):
        if k + DEPTH - 1 < NCH3:
            handles[(k + DEPTH - 1) % DEPTH] = start(k + DEPTH - 1)
        handles[k % DEPTH].wait()
        pltpu.sync_copy(bufs[k % DEPTH],
                        out_hbm.at[b, pl.ds(base + k * CH, CH)])


_RB = 128


def _pad_body(x_ref, o_ref):
    i = pl.program_id(1)

    @pl.when(i < M // _RB)
    def _copy():
        o_ref[...] = x_ref[...]

    @pl.when(i >= M // _RB)
    def _zero():
        o_ref[...] = jnp.zeros_like(o_ref)


_pad_tc = pl.pallas_call(
    _pad_body,
    grid=(B, MP // _RB),
    in_specs=[pl.BlockSpec((1, _RB, F),
                           lambda b, i: (b, jnp.minimum(i, M // _RB - 1), 0))],
    out_specs=pl.BlockSpec((1, _RB, F), lambda b, i: (b, i, 0)),
    out_shape=jax.ShapeDtypeStruct((B, MP, F), jnp.float32),
)


def kernel(feats, interpolate_idx, upsample_idx):
    assert feats.shape == (B, M, F) and upsample_idx.shape == (B, N)
    feats_ext = _pad_tc(feats).reshape(B * MP, F)
    return _upsample_sc(feats_ext, interpolate_idx.astype(jnp.int32),
                        upsample_idx.astype(jnp.int32))


# speedup vs baseline: 11.8224x; 2.3225x over previous
import functools
import jax
import jax.numpy as jnp
from jax import lax
from jax.experimental import pallas as pl
from jax.experimental.pallas import tpu as pltpu
from jax.experimental.pallas import tpu_sc as plsc

B, M, N, F = 4, 16384, 32768, 256
PAD = 128
MP = M + PAD
NSUB = 16
WPB = 8
JW = N // WPB
NCH1 = JW // 16
CH = 32
DEPTH = 3
NCH3 = JW // CH

_mesh = plsc.VectorSubcoreMesh(core_axis_name="c", subcore_axis_name="s")


@functools.partial(
    pl.kernel,
    out_type=jax.ShapeDtypeStruct((B, N, F), jnp.float32),
    mesh=_mesh,
    compiler_params=pltpu.CompilerParams(needs_layout_passes=False),
    scratch_types=[
        pltpu.VMEM((N,), jnp.int32),
        pltpu.VMEM((M,), jnp.int32),
        pltpu.VMEM((JW,), jnp.int32),
        pltpu.VMEM((4, JW), jnp.int32),
        pltpu.VMEM_SHARED((NSUB, N), jnp.int32),
        pltpu.VMEM((64,), jnp.int32),
        [pltpu.VMEM((CH, F), jnp.float32)] * DEPTH,
        [pltpu.SemaphoreType.DMA] * DEPTH,
    ],
)
def _upsample_sc(feats_hbm, interp_hbm, up_hbm, out_hbm,
                 pos_v, interp_v, acc_v, tmp4, shared,
                 nbr_v, bufs, sems):
    c = lax.axis_index("c")
    s = lax.axis_index("s")
    b = 2 * c + s // WPB
    r = s % WPB
    base = r * JW

    up_dma = pltpu.async_copy(up_hbm.at[b, pl.ds(base, JW)], tmp4.at[3],
                              sems[0])
    in_dma = pltpu.async_copy(interp_hbm.at[b], interp_v, sems[1])
    m1 = jnp.full((16,), -1, jnp.int32)

    def ms(i, carry):
        for u in range(8):
            pos_v[pl.ds(i * 128 + u * 16, 16)] = m1
        return carry

    lax.fori_loop(0, N // 128, ms, 0)
    up_dma.wait()
    in_dma.wait()

    ii = lax.iota(jnp.int32, 16)
    nbr_v[pl.ds(16, 16)] = jnp.full((16,), -1, jnp.int32)
    nbr_v[pl.ds(48, 16)] = jnp.full((16,), -1, jnp.int32)

    def ph1(ci, carry):
        res = []
        for u in range(2):
            idx = tmp4[3, pl.ds(ci * 32 + u * 16, 16)]
            nvec = base + ci * 32 + u * 16 + ii
            comb = (idx << 15) | nvec
            scomb, n_s = plsc.sort_key_val(comb, nvec)
            idx_s = lax.shift_right_logical(scomb, 15)
            nbr_v[pl.ds(u * 32, 16)] = idx_s
            nxt = nbr_v[pl.ds(u * 32 + 1, 16)]
            res.append((idx_s, n_s, idx_s != nxt))
        for idx_s, n_s, is_last in res:
            plsc.store_scatter(pos_v, [idx_s], n_s, mask=is_last)
        return carry

    lax.fori_loop(0, NCH1 // 2, ph1, 0)

    pltpu.sync_copy(pos_v, shared.at[s])
    plsc.subcore_barrier()

    g0 = (s // WPB) * WPB
    mh = [pltpu.async_copy(shared.at[g0 + k, pl.ds(base, JW)],
                           tmp4.at[k], sems[k % DEPTH])
          for k in range(4)]
    for h in mh:
        h.wait()

    def mg1(i, carry):
        sl = pl.ds(i * 16, 16)
        pos = jnp.maximum(jnp.maximum(tmp4[0, sl], tmp4[1, sl]),
                          jnp.maximum(tmp4[2, sl], tmp4[3, sl]))
        acc_v[sl] = pos
        return carry

    lax.fori_loop(0, JW // 16, mg1, 0)

    mh = [pltpu.async_copy(shared.at[g0 + 4 + k, pl.ds(base, JW)],
                           tmp4.at[k], sems[k % DEPTH])
          for k in range(4)]
    for h in mh:
        h.wait()

    boff = b * MP

    def mp(i, carry):
        sl = pl.ds(i * 16, 16)
        pos = jnp.maximum(jnp.maximum(tmp4[0, sl], tmp4[1, sl]),
                          jnp.maximum(tmp4[2, sl], tmp4[3, sl]))
        pos = jnp.maximum(pos, acc_v[sl])
        cidx = jnp.maximum(pos - M, 0)
        ival = plsc.load_gather(interp_v, [cidx])
        row = jnp.where(pos >= M, ival, pos)
        zrow = M + ((i * 16 + ii) & (PAD - 1))
        row = jnp.where(pos >= 0, row, zrow)
        acc_v[sl] = row + boff
        return carry

    lax.fori_loop(0, JW // 16, mp, 0)

    handles = [None] * DEPTH

    def start(k):
        idx_ref = acc_v.at[pl.ds(k * CH, CH)]
        return pltpu.async_copy(feats_hbm.at[idx_ref],
                                bufs[k % DEPTH], sems[k % DEPTH])

    for k in range(DEPTH - 1):
        handles[k] = start(k)
    for k in range(NCH3):
        if k + DEPTH - 1 < NCH3:
            handles[(k + DEPTH - 1) % DEPTH] = start(k + DEPTH - 1)
        handles[k % DEPTH].wait()
        pltpu.sync_copy(bufs[k % DEPTH],
                        out_hbm.at[b, pl.ds(base + k * CH, CH)])


def kernel(feats, interpolate_idx, upsample_idx):
    assert feats.shape == (B, M, F) and upsample_idx.shape == (B, N)
    zrows = jnp.zeros((B, PAD, F), jnp.float32)
    feats_ext = jnp.concatenate([feats, zrows], axis=1).reshape(B * MP, F)
    return _upsample_sc(feats_ext, interpolate_idx.astype(jnp.int32),
                        upsample_idx.astype(jnp.int32))
